# baseline (speedup 1.0000x reference)
import functools
import math

import jax
import jax.numpy as jnp
from jax import lax
from jax.experimental import pallas as pl
from jax.experimental.pallas import tpu as pltpu
from jax.experimental.pallas import tpu_sc as plsc

_PAD = 1
_SMOOTH = 0.1

_info = plsc.get_sparse_core_info()
_NC, _NS, _L = _info.num_cores, _info.num_subcores, _info.num_lanes
_NW = _NC * _NS

_N_TC = 1280


def _make_sc_kernel(N, V):
    rpw = (N - _N_TC) // _NW
    n_un = 20
    n_acc = 8
    mesh = plsc.VectorSubcoreMesh(core_axis_name="c", subcore_axis_name="s")

    @functools.partial(
        pl.kernel,
        mesh=mesh,
        out_type=[
            jax.ShapeDtypeStruct((_NW, 128), jnp.float32),
            jax.ShapeDtypeStruct((_NW * rpw * _L,), jnp.float32),
        ],
        scratch_types=[
            pltpu.VMEM((rpw * _L,), jnp.int32),
            pltpu.VMEM((rpw * _L,), jnp.int32),
            pltpu.VMEM((1, V), jnp.float32),
            pltpu.VMEM((1, V), jnp.float32),
            pltpu.VMEM((rpw * _L,), jnp.float32),
            pltpu.VMEM((128,), jnp.float32),
            pltpu.SemaphoreType.DMA,
            pltpu.SemaphoreType.DMA,
        ],
    )
    def k(lp2d, t_hbm, out_hbm, out2_hbm,
          idxe_v, tb_v, buf0, buf1, rows_v, res_v, sem0, sem1):
        wid = lax.axis_index("s") * _NC + lax.axis_index("c")
        dbase = _N_TC + wid * rpw
        zero = jnp.zeros((_L,), jnp.float32)
        izero = jnp.zeros((_L,), jnp.int32)
        lane = lax.broadcasted_iota(jnp.int32, (_L,), 0)

        for r in range(rpw):
            idxe_v[pl.ds(r * _L, _L)] = izero + (dbase + r)
        gds = []
        for g in range(rpw * _L // 128):
            gds.append(pltpu.async_copy(
                t_hbm.at[idxe_v.at[pl.ds(g * 128, 128)]],
                tb_v.at[pl.ds(g * 128, 128)],
                sem0,
            ))
        for d in gds:
            d.wait()

        bufs = (buf0, buf1)
        sems = (sem0, sem1)
        pend = [None, None]
        pend[0] = pltpu.async_copy(lp2d.at[pl.ds(dbase, 1)], buf0, sems[0])
        acc_t = zero
        acc_p = zero
        for rr in range(rpw):
            b = rr % 2
            if rr + 1 < rpw:
                nb = (rr + 1) % 2
                pend[nb] = pltpu.async_copy(
                    lp2d.at[pl.ds(dbase + rr + 1, 1)], bufs[nb], sems[nb]
                )
            pend[b].wait()
            buf = bufs[b]

            def body(i, carry, buf=buf, rr=rr):
                accs = list(carry[:n_acc])
                tacc = carry[n_acc]
                swordf = lax.shift_right_logical(
                    tb_v[pl.ds(rr * _L, _L)], 4
                ).astype(jnp.float32)
                off = i * (n_un * _L)
                sbase = (i * n_un).astype(jnp.float32) - swordf
                for u in range(n_un):
                    sl = buf[0, pl.ds(off + u * _L, _L)]
                    accs[u % n_acc] = accs[u % n_acc] + sl
                    wg = jnp.maximum(1.0 - jnp.abs(sbase + float(u)), 0.0)
                    tacc = tacc + sl * wg
                return tuple(accs) + (tacc,)

            carry = lax.fori_loop(
                0, V // (n_un * _L), body, (zero,) * (n_acc + 1), unroll=False
            )
            row = carry[0]
            for a in carry[1:n_acc]:
                row = row + a
            rows_v[pl.ds(rr * _L, _L)] = row
            tvec = tb_v[pl.ds(rr * _L, _L)]
            tvf = tvec.astype(jnp.float32)
            mvec = jnp.minimum(jnp.abs(tvf - float(_PAD)), 1.0)
            lanef_o = lax.broadcasted_iota(jnp.int32, (_L,), 0).astype(
                jnp.float32
            )
            tlanef = jnp.bitwise_and(tvec, _L - 1).astype(jnp.float32)
            ohT = jnp.maximum(1.0 - jnp.abs(lanef_o - tlanef), 0.0)
            oh1 = jnp.maximum(1.0 - jnp.abs(lanef_o - float(_PAD)), 0.0)
            psl = buf[0, pl.ds(0, _L)]
            acc_t = acc_t + carry[n_acc] * ohT * mvec
            acc_p = acc_p + psl * oh1 * mvec

        res_v[pl.ds(0, _L)] = acc_t
        res_v[pl.ds(_L, _L)] = acc_p
        for s in range(2, 128 // _L):
            res_v[pl.ds(s * _L, _L)] = zero
        pltpu.sync_copy(res_v, out_hbm.at[wid])
        pltpu.sync_copy(rows_v, out2_hbm.at[pl.ds(wid * rpw * _L, rpw * _L)])

    return k


def _make_tc_sum(V, rb):
    nsteps = _N_TC // rb

    def body(t_ref, lp_ref, out_ref):
        i = pl.program_id(0)
        t = t_ref[0, 0, :].reshape(rb, 1)
        m = t != _PAD
        lp = lp_ref[...]
        cols = lax.broadcasted_iota(jnp.int32, (rb, V), 1)
        masked = jnp.where(m, lp, 0.0)
        s_part = jnp.sum(masked)
        t_part = jnp.sum(jnp.where(cols == t, masked, 0.0))
        p_part = jnp.sum(jnp.where(cols == _PAD, masked, 0.0))
        c_part = jnp.sum(m.astype(jnp.float32))

        @pl.when(i == 0)
        def _():
            out_ref[0, 0] = s_part
            out_ref[0, 1] = t_part
            out_ref[0, 2] = p_part
            out_ref[0, 3] = c_part

        @pl.when(i != 0)
        def _():
            out_ref[0, 0] += s_part
            out_ref[0, 1] += t_part
            out_ref[0, 2] += p_part
            out_ref[0, 3] += c_part

    return pl.pallas_call(
        body,
        grid=(nsteps,),
        in_specs=[
            pl.BlockSpec((1, 1, rb), lambda i: (i, 0, 0)),
            pl.BlockSpec((rb, V), lambda i: (i, 0)),
        ],
        out_specs=pl.BlockSpec(memory_space=pltpu.SMEM),
        out_shape=jax.ShapeDtypeStruct((1, 4), jnp.float32),
    )


def _make_combine(N, eps, centropy):
    nblk = (N - _N_TC) // 128

    def body(s_ref, g_ref, rows_ref, t_ref, out_ref):
        g = g_ref[...]
        lane = lax.broadcasted_iota(jnp.int32, (_NW, 128), 1)
        coef = jnp.where(
            lane < _L,
            eps - (1.0 - _SMOOTH),
            jnp.where(lane < 2 * _L, eps, 0.0),
        )
        t2 = t_ref[...].reshape(nblk, 128)
        m2 = t2 != _PAD
        srow = jnp.sum(rows_ref[...], axis=2)
        s_sc = jnp.sum(jnp.where(m2, srow, 0.0))
        cnt_sc = jnp.sum(m2.astype(jnp.float32))
        s_tot = s_ref[0, 0] + s_sc
        t_tot = s_ref[0, 1]
        p_tot = s_ref[0, 2]
        cnt_tot = s_ref[0, 3] + cnt_sc
        out_ref[0, 0] = (
            cnt_tot * centropy
            - eps * s_tot
            + (eps - (1.0 - _SMOOTH)) * t_tot
            + eps * p_tot
            + jnp.sum(g * coef)
        )

    return pl.pallas_call(
        body,
        in_specs=[
            pl.BlockSpec(memory_space=pltpu.SMEM),
            pl.BlockSpec((_NW, 128), lambda: (0, 0)),
            pl.BlockSpec((nblk, 128, _L), lambda: (0, 0, 0)),
            pl.BlockSpec((nblk, 1, 128), lambda: (0, 0, 0)),
        ],
        out_specs=pl.BlockSpec(memory_space=pltpu.SMEM),
        out_shape=jax.ShapeDtypeStruct((1, 1), jnp.float32),
    )


def kernel(log_probs, targets):
    b, s, v = log_probs.shape
    n = b * s
    eps = _SMOOTH / (v - 2)
    centropy = (1.0 - _SMOOTH) * math.log(1.0 - _SMOOTH) + _SMOOTH * math.log(eps)

    t_flat = targets.reshape(n).astype(jnp.int32)
    lp2 = log_probs.reshape(n, v)
    partials, rowsums = _make_sc_kernel(n, v)(lp2, t_flat)

    rb = 128
    t3 = t_flat[:_N_TC].reshape(_N_TC // rb, 1, rb)
    stpc = _make_tc_sum(v, rb)(t3, lp2)

    nblk = (n - _N_TC) // 128
    rows3 = rowsums.reshape(nblk, 128, _L)
    tsc3 = t_flat[_N_TC:].reshape(nblk, 1, 128)
    out = _make_combine(n, eps, centropy)(stpc, partials, rows3, tsc3)
    return out[0, 0]


# speedup vs baseline: 1.0308x; 1.0308x over previous
import functools
import math

import jax
import jax.numpy as jnp
from jax import lax
from jax.experimental import pallas as pl
from jax.experimental.pallas import tpu as pltpu
from jax.experimental.pallas import tpu_sc as plsc

_PAD = 1
_SMOOTH = 0.1

_info = plsc.get_sparse_core_info()
_NC, _NS, _L = _info.num_cores, _info.num_subcores, _info.num_lanes
_NW = _NC * _NS

_N_TC = 1280


def _make_sc_kernel(N, V):
    rpw = (N - _N_TC) // _NW
    n_un = 16
    n_acc = 8
    mesh = plsc.VectorSubcoreMesh(core_axis_name="c", subcore_axis_name="s")

    @functools.partial(
        pl.kernel,
        mesh=mesh,
        out_type=[
            jax.ShapeDtypeStruct((_NW, 128), jnp.float32),
            jax.ShapeDtypeStruct((_NW * rpw * _L,), jnp.float32),
        ],
        scratch_types=[
            pltpu.VMEM((rpw * _L,), jnp.int32),
            pltpu.VMEM((rpw * _L,), jnp.int32),
            pltpu.VMEM((1, V), jnp.float32),
            pltpu.VMEM((1, V), jnp.float32),
            pltpu.VMEM((rpw * _L,), jnp.float32),
            pltpu.VMEM((128,), jnp.float32),
            pltpu.SemaphoreType.DMA,
            pltpu.SemaphoreType.DMA,
        ],
    )
    def k(lp2d, t_hbm, out_hbm, out2_hbm,
          idxe_v, tb_v, buf0, buf1, rows_v, res_v, sem0, sem1):
        wid = lax.axis_index("s") * _NC + lax.axis_index("c")
        dbase = _N_TC + wid * rpw
        zero = jnp.zeros((_L,), jnp.float32)
        izero = jnp.zeros((_L,), jnp.int32)
        lane = lax.broadcasted_iota(jnp.int32, (_L,), 0)

        for r in range(rpw):
            idxe_v[pl.ds(r * _L, _L)] = izero + (dbase + r)
        gds = []
        for g in range(rpw * _L // 128):
            gds.append(pltpu.async_copy(
                t_hbm.at[idxe_v.at[pl.ds(g * 128, 128)]],
                tb_v.at[pl.ds(g * 128, 128)],
                sem0,
            ))
        for d in gds:
            d.wait()

        bufs = (buf0, buf1)
        sems = (sem0, sem1)
        pend = [None, None]
        pend[0] = pltpu.async_copy(lp2d.at[pl.ds(dbase, 1)], buf0, sems[0])
        acc_t = zero
        acc_p = zero
        for rr in range(rpw):
            b = rr % 2
            if rr + 1 < rpw:
                nb = (rr + 1) % 2
                pend[nb] = pltpu.async_copy(
                    lp2d.at[pl.ds(dbase + rr + 1, 1)], bufs[nb], sems[nb]
                )
            pend[b].wait()
            buf = bufs[b]

            def body(i, carry, buf=buf, rr=rr):
                accs = list(carry[:n_acc])
                tacc = carry[n_acc]
                swordf = lax.shift_right_logical(
                    tb_v[pl.ds(rr * _L, _L)], 4
                ).astype(jnp.float32)
                off = i * (n_un * _L)
                sbase = (i * n_un).astype(jnp.float32) - swordf
                for u in range(n_un):
                    sl = buf[0, pl.ds(off + u * _L, _L)]
                    accs[u % n_acc] = accs[u % n_acc] + sl
                    wg = jnp.maximum(1.0 - jnp.abs(sbase + float(u)), 0.0)
                    tacc = tacc + sl * wg
                return tuple(accs) + (tacc,)

            carry = lax.fori_loop(
                0, V // (n_un * _L), body, (zero,) * (n_acc + 1), unroll=False
            )
            row = carry[0]
            for a in carry[1:n_acc]:
                row = row + a
            rows_v[pl.ds(rr * _L, _L)] = row
            tvec = tb_v[pl.ds(rr * _L, _L)]
            tvf = tvec.astype(jnp.float32)
            mvec = jnp.minimum(jnp.abs(tvf - float(_PAD)), 1.0)
            lanef_o = lax.broadcasted_iota(jnp.int32, (_L,), 0).astype(
                jnp.float32
            )
            tlanef = jnp.bitwise_and(tvec, _L - 1).astype(jnp.float32)
            ohT = jnp.maximum(1.0 - jnp.abs(lanef_o - tlanef), 0.0)
            oh1 = jnp.maximum(1.0 - jnp.abs(lanef_o - float(_PAD)), 0.0)
            psl = buf[0, pl.ds(0, _L)]
            acc_t = acc_t + carry[n_acc] * ohT * mvec
            acc_p = acc_p + psl * oh1 * mvec

        res_v[pl.ds(0, _L)] = acc_t
        res_v[pl.ds(_L, _L)] = acc_p
        for s in range(2, 128 // _L):
            res_v[pl.ds(s * _L, _L)] = zero
        pltpu.sync_copy(res_v, out_hbm.at[wid])
        pltpu.sync_copy(rows_v, out2_hbm.at[pl.ds(wid * rpw * _L, rpw * _L)])

    return k


def _make_tc_sum(V, rb):
    nsteps = _N_TC // rb

    def body(t_ref, lp_ref, out_ref):
        i = pl.program_id(0)
        t = t_ref[0, 0, :].reshape(rb, 1)
        m = t != _PAD
        lp = lp_ref[...]
        cols = lax.broadcasted_iota(jnp.int32, (rb, V), 1)
        masked = jnp.where(m, lp, 0.0)
        s_part = jnp.sum(masked)
        t_part = jnp.sum(jnp.where(cols == t, masked, 0.0))
        p_part = jnp.sum(jnp.where(cols == _PAD, masked, 0.0))
        c_part = jnp.sum(m.astype(jnp.float32))

        @pl.when(i == 0)
        def _():
            out_ref[0, 0] = s_part
            out_ref[0, 1] = t_part
            out_ref[0, 2] = p_part
            out_ref[0, 3] = c_part

        @pl.when(i != 0)
        def _():
            out_ref[0, 0] += s_part
            out_ref[0, 1] += t_part
            out_ref[0, 2] += p_part
            out_ref[0, 3] += c_part

    return pl.pallas_call(
        body,
        grid=(nsteps,),
        in_specs=[
            pl.BlockSpec((1, 1, rb), lambda i: (i, 0, 0)),
            pl.BlockSpec((rb, V), lambda i: (i, 0)),
        ],
        out_specs=pl.BlockSpec(memory_space=pltpu.SMEM),
        out_shape=jax.ShapeDtypeStruct((1, 4), jnp.float32),
    )


def _make_combine(N, eps, centropy):
    nblk = (N - _N_TC) // 128

    def body(s_ref, g_ref, rows_ref, t_ref, out_ref):
        g = g_ref[...]
        lane = lax.broadcasted_iota(jnp.int32, (_NW, 128), 1)
        coef = jnp.where(
            lane < _L,
            eps - (1.0 - _SMOOTH),
            jnp.where(lane < 2 * _L, eps, 0.0),
        )
        t2 = t_ref[...].reshape(nblk, 128)
        m2 = t2 != _PAD
        srow = jnp.sum(rows_ref[...], axis=2)
        s_sc = jnp.sum(jnp.where(m2, srow, 0.0))
        cnt_sc = jnp.sum(m2.astype(jnp.float32))
        s_tot = s_ref[0, 0] + s_sc
        t_tot = s_ref[0, 1]
        p_tot = s_ref[0, 2]
        cnt_tot = s_ref[0, 3] + cnt_sc
        out_ref[0, 0] = (
            cnt_tot * centropy
            - eps * s_tot
            + (eps - (1.0 - _SMOOTH)) * t_tot
            + eps * p_tot
            + jnp.sum(g * coef)
        )

    return pl.pallas_call(
        body,
        in_specs=[
            pl.BlockSpec(memory_space=pltpu.SMEM),
            pl.BlockSpec((_NW, 128), lambda: (0, 0)),
            pl.BlockSpec((nblk, 128, _L), lambda: (0, 0, 0)),
            pl.BlockSpec((nblk, 1, 128), lambda: (0, 0, 0)),
        ],
        out_specs=pl.BlockSpec(memory_space=pltpu.SMEM),
        out_shape=jax.ShapeDtypeStruct((1, 1), jnp.float32),
    )


def kernel(log_probs, targets):
    b, s, v = log_probs.shape
    n = b * s
    eps = _SMOOTH / (v - 2)
    centropy = (1.0 - _SMOOTH) * math.log(1.0 - _SMOOTH) + _SMOOTH * math.log(eps)

    t_flat = targets.reshape(n).astype(jnp.int32)
    lp2 = log_probs.reshape(n, v)
    partials, rowsums = _make_sc_kernel(n, v)(lp2, t_flat)

    rb = 128
    t3 = t_flat[:_N_TC].reshape(_N_TC // rb, 1, rb)
    stpc = _make_tc_sum(v, rb)(t3, lp2)

    nblk = (n - _N_TC) // 128
    rows3 = rowsums.reshape(nblk, 128, _L)
    tsc3 = t_flat[_N_TC:].reshape(nblk, 1, 128)
    out = _make_combine(n, eps, centropy)(stpc, partials, rows3, tsc3)
    return out[0, 0]
